# baseline (speedup 1.0000x reference)
import functools

import jax
import jax.numpy as jnp
from jax import lax
from jax.experimental import pallas as pl
from jax.experimental.pallas import tpu as pltpu, tpu_sc as plsc

_B = 16384
_D = 32
_V = 1000000
_NW = 32
_L = 16

_NPT = 100
_VSC = _NW * _NPT * 128
_CPW = _NPT * 128
_BLK = 51200
_TC_OFF = _VSC // _BLK
_TC_GRID = (_V - _VSC + _BLK - 1) // _BLK
_VTC = _TC_GRID * _BLK

_BPW = _B // _NW
_CHUNK = 128
_NCH = _BPW // _CHUNK

_sc_mesh = plsc.VectorSubcoreMesh(core_axis_name="c", subcore_axis_name="s")


@functools.partial(
    pl.kernel,
    mesh=_sc_mesh,
    out_type=jax.ShapeDtypeStruct((_VSC,), jnp.float32),
    compiler_params=pltpu.CompilerParams(
        needs_layout_passes=False, use_tc_tiling_on_sc=True
    ),
    scratch_types=[
        pltpu.VMEM((_D, 128), jnp.float32),
        pltpu.VMEM((_D, 128), jnp.float32),
        pltpu.VMEM((_D,), jnp.float32),
        pltpu.VMEM((_CPW,), jnp.float32),
        pltpu.SemaphoreType.DMA,
        pltpu.SemaphoreType.DMA,
    ],
)
def _matvec_sc(tt_hbm, user_hbm, s_hbm, st0, st1, u_v, s_buf, sem0, sem1):
    wid = lax.axis_index("s") * 2 + lax.axis_index("c")
    base = wid * _CPW

    pltpu.sync_copy(user_hbm, u_v)
    u_lo = u_v[pl.ds(0, _L)]
    u_hi = u_v[pl.ds(_L, _L)]
    u = [u_lo[d] for d in range(_L)] + [u_hi[d] for d in range(_L)]

    def _col(p):
        return pl.multiple_of(base + p * 128, 128)

    def _fire(p, st, sem):
        pltpu.async_copy(tt_hbm.at[:, pl.ds(_col(p), 128)], st, sem)

    def _wait(st, sem):
        pltpu.make_async_copy(
            tt_hbm.at[:, pl.ds(0, 128)], st, sem
        ).wait()

    def _compute(p, st):
        accs = [jnp.zeros((_L,), jnp.float32) for _ in range(8)]
        for d in range(_D):
            ud = u[d]
            for h in range(8):
                accs[h] = accs[h] + st[d, pl.ds(16 * h, _L)] * ud
        p0 = pl.multiple_of(p * 128, 128)
        for h in range(8):
            s_buf[pl.ds(p0 + 16 * h, _L)] = accs[h]

    _fire(0, st0, sem0)
    _fire(1, st1, sem1)

    def body(g, carry):
        p = g * 2
        _wait(st0, sem0)
        _compute(p, st0)

        @pl.when(g < _NPT // 2 - 1)
        def _():
            _fire(p + 2, st0, sem0)

        _wait(st1, sem1)
        _compute(p + 1, st1)

        @pl.when(g < _NPT // 2 - 1)
        def _():
            _fire(p + 3, st1, sem1)

        return carry

    lax.fori_loop(0, _NPT // 2, body, 0)

    pltpu.sync_copy(s_buf, s_hbm.at[pl.ds(base, _CPW)])


def _matvec_tc_body(u_ref, t_ref, s_ref):
    s_ref[...] = jnp.dot(
        u_ref[...], t_ref[...], preferred_element_type=jnp.float32
    ).reshape(_BLK)


def _matvec_tc(tt, user):
    return pl.pallas_call(
        _matvec_tc_body,
        grid=(_TC_GRID,),
        in_specs=[
            pl.BlockSpec((1, _D), lambda i: (0, 0)),
            pl.BlockSpec((_D, _BLK), lambda i: (0, i + _TC_OFF)),
        ],
        out_specs=pl.BlockSpec((_BLK,), lambda i: (i,)),
        out_shape=jax.ShapeDtypeStruct((_VTC,), jnp.float32),
    )(user, tt)


@functools.partial(
    pl.kernel,
    mesh=_sc_mesh,
    out_type=jax.ShapeDtypeStruct((_B,), jnp.float32),
    compiler_params=pltpu.CompilerParams(
        needs_layout_passes=False, use_tc_tiling_on_sc=False
    ),
    scratch_types=[
        pltpu.VMEM((_NCH, _CHUNK), jnp.int32),
        pltpu.VMEM((_NCH, _CHUNK), jnp.int32),
        pltpu.VMEM((_NCH, _CHUNK), jnp.int32),
        pltpu.VMEM((_BPW,), jnp.float32),
        pltpu.VMEM((_BPW,), jnp.float32),
        pltpu.VMEM((_BPW,), jnp.float32),
        pltpu.SemaphoreType.DMA,
    ],
)
def _gather_sigmoid(idx_hbm, ssc_hbm, stc_hbm, out_hbm,
                    idx_v, lo_v, hi_v, a_v, b_v, out_v, sem):
    wid = lax.axis_index("s") * 2 + lax.axis_index("c")
    base = wid * _BPW

    pltpu.sync_copy(idx_hbm.at[pl.ds(wid * _NCH, _NCH)], idx_v)

    for j in range(_NCH):
        for m in range(_CHUNK // _L):
            r = idx_v[j, pl.ds(_L * m, _L)]
            in_sc = r < _VSC
            lo = jnp.where(in_sc, r,
                           jnp.where(r < 2 * _VSC, r - _VSC, r - 2 * _VSC))
            hi = jnp.where(in_sc, r, r - _VSC)
            lo_v[j, pl.ds(_L * m, _L)] = lo
            hi_v[j, pl.ds(_L * m, _L)] = hi

    copies = []
    for j in range(_NCH):
        copies.append(pltpu.async_copy(
            ssc_hbm.at[lo_v.at[j]], a_v.at[pl.ds(j * _CHUNK, _CHUNK)], sem))
        copies.append(pltpu.async_copy(
            stc_hbm.at[hi_v.at[j]], b_v.at[pl.ds(j * _CHUNK, _CHUNK)], sem))
    for c in copies:
        c.wait()

    for g in range(_BPW // _L):
        r0 = g * _L
        j = g // (_CHUNK // _L)
        m = g % (_CHUNK // _L)
        r = idx_v[j, pl.ds(_L * m, _L)]
        x = jnp.where(r < _VSC, a_v[pl.ds(r0, _L)], b_v[pl.ds(r0, _L)])
        out_v[pl.ds(r0, _L)] = 1.0 / (1.0 + jnp.exp(-x))

    pltpu.sync_copy(out_v, out_hbm.at[pl.ds(base, _BPW)])


def kernel(item_indices, item_table, user_table):
    tt = item_table.T
    user = user_table.reshape(_D)
    s_sc = _matvec_sc(tt, user)
    s_tc = _matvec_tc(tt, user_table)
    idx = item_indices.astype(jnp.int32).reshape(_NW * _NCH, _CHUNK)
    return _gather_sigmoid(idx, s_sc, s_tc)


# speedup vs baseline: 1.5211x; 1.5211x over previous
import functools

import jax
import jax.numpy as jnp
from jax import lax
from jax.experimental import pallas as pl
from jax.experimental.pallas import tpu as pltpu, tpu_sc as plsc

_B = 16384
_D = 32
_V = 1000000
_NW = 32
_L = 16

_NPT = 64
_VSC = _NW * _NPT * 128
_CPW = _NPT * 128
_BLK = 65536
_TC_OFF = _VSC // _BLK
_TC_GRID = (_V - _VSC + _BLK - 1) // _BLK
_VTC = _TC_GRID * _BLK
_NBUF = 4

_BPW = _B // _NW
_CHUNK = 128
_NCH = _BPW // _CHUNK

_sc_mesh = plsc.VectorSubcoreMesh(core_axis_name="c", subcore_axis_name="s")


@functools.partial(
    pl.kernel,
    mesh=_sc_mesh,
    out_type=jax.ShapeDtypeStruct((_VSC,), jnp.float32),
    compiler_params=pltpu.CompilerParams(
        needs_layout_passes=False, use_tc_tiling_on_sc=True
    ),
    scratch_types=[
        pltpu.VMEM((_NBUF, _D, 128), jnp.float32),
        pltpu.VMEM((_D,), jnp.float32),
        pltpu.VMEM((_CPW,), jnp.float32),
        pltpu.SemaphoreType.DMA,
        pltpu.SemaphoreType.DMA,
        pltpu.SemaphoreType.DMA,
        pltpu.SemaphoreType.DMA,
    ],
)
def _matvec_sc(tt_hbm, user_hbm, s_hbm, stages, u_v, s_buf, *sems):
    wid = lax.axis_index("s") * 2 + lax.axis_index("c")
    base = wid * _CPW

    pltpu.sync_copy(user_hbm, u_v)
    u_lo = u_v[pl.ds(0, _L)]
    u_hi = u_v[pl.ds(_L, _L)]
    u = [u_lo[d] for d in range(_L)] + [u_hi[d] for d in range(_L)]

    def _col(p):
        return pl.multiple_of(base + p * 128, 128)

    def _fire(p, b):
        pltpu.async_copy(
            tt_hbm.at[:, pl.ds(_col(p), 128)], stages.at[b], sems[b]
        )

    def _wait(b):
        pltpu.make_async_copy(
            tt_hbm.at[:, pl.ds(0, 128)], stages.at[b], sems[b]
        ).wait()

    def _compute(p, b):
        st = stages.at[b]
        p0 = pl.multiple_of(p * 128, 128)
        for h in range(8):
            a0 = jnp.zeros((_L,), jnp.float32)
            a1 = jnp.zeros((_L,), jnp.float32)
            for d in range(0, _D, 2):
                a0 = a0 + st[d, pl.ds(16 * h, _L)] * u[d]
                a1 = a1 + st[d + 1, pl.ds(16 * h, _L)] * u[d + 1]
            s_buf[pl.ds(p0 + 16 * h, _L)] = a0 + a1

    for b in range(_NBUF):
        _fire(b, b)

    def body(g, carry):
        p = g * _NBUF
        for b in range(_NBUF):
            _wait(b)
            _compute(p + b, b)

            @pl.when(g < _NPT // _NBUF - 1)
            def _():
                _fire(p + b + _NBUF, b)

        return carry

    lax.fori_loop(0, _NPT // _NBUF, body, 0)

    pltpu.sync_copy(s_buf, s_hbm.at[pl.ds(base, _CPW)])


def _matvec_tc_body(u_ref, t_ref, s_ref):
    s_ref[...] = jnp.dot(
        u_ref[...], t_ref[...], preferred_element_type=jnp.float32
    ).reshape(_BLK)


def _matvec_tc(tt, user):
    return pl.pallas_call(
        _matvec_tc_body,
        grid=(_TC_GRID,),
        in_specs=[
            pl.BlockSpec((1, _D), lambda i: (0, 0)),
            pl.BlockSpec((_D, _BLK), lambda i: (0, i + _TC_OFF)),
        ],
        out_specs=pl.BlockSpec((_BLK,), lambda i: (i,)),
        out_shape=jax.ShapeDtypeStruct((_VTC,), jnp.float32),
    )(user, tt)


@functools.partial(
    pl.kernel,
    mesh=_sc_mesh,
    out_type=jax.ShapeDtypeStruct((_B,), jnp.float32),
    compiler_params=pltpu.CompilerParams(
        needs_layout_passes=False, use_tc_tiling_on_sc=False
    ),
    scratch_types=[
        pltpu.VMEM((_NCH, _CHUNK), jnp.int32),
        pltpu.VMEM((_NCH, _CHUNK), jnp.int32),
        pltpu.VMEM((_NCH, _CHUNK), jnp.int32),
        pltpu.VMEM((_BPW,), jnp.float32),
        pltpu.VMEM((_BPW,), jnp.float32),
        pltpu.VMEM((_BPW,), jnp.float32),
        pltpu.SemaphoreType.DMA,
    ],
)
def _gather_sigmoid(idx_hbm, ssc_hbm, stc_hbm, out_hbm,
                    idx_v, lo_v, hi_v, a_v, b_v, out_v, sem):
    wid = lax.axis_index("s") * 2 + lax.axis_index("c")
    base = wid * _BPW

    pltpu.sync_copy(idx_hbm.at[pl.ds(wid * _NCH, _NCH)], idx_v)

    for j in range(_NCH):
        for m in range(_CHUNK // _L):
            r = idx_v[j, pl.ds(_L * m, _L)]
            in_sc = r < _VSC
            lo = jnp.where(in_sc, r,
                           jnp.where(r < 2 * _VSC, r - _VSC, r - 2 * _VSC))
            hi = jnp.where(in_sc, r, r - _VSC)
            lo_v[j, pl.ds(_L * m, _L)] = lo
            hi_v[j, pl.ds(_L * m, _L)] = hi

    copies = []
    for j in range(_NCH):
        copies.append(pltpu.async_copy(
            ssc_hbm.at[lo_v.at[j]], a_v.at[pl.ds(j * _CHUNK, _CHUNK)], sem))
        copies.append(pltpu.async_copy(
            stc_hbm.at[hi_v.at[j]], b_v.at[pl.ds(j * _CHUNK, _CHUNK)], sem))
    for c in copies:
        c.wait()

    for g in range(_BPW // _L):
        r0 = g * _L
        j = g // (_CHUNK // _L)
        m = g % (_CHUNK // _L)
        r = idx_v[j, pl.ds(_L * m, _L)]
        x = jnp.where(r < _VSC, a_v[pl.ds(r0, _L)], b_v[pl.ds(r0, _L)])
        out_v[pl.ds(r0, _L)] = 1.0 / (1.0 + jnp.exp(-x))

    pltpu.sync_copy(out_v, out_hbm.at[pl.ds(base, _BPW)])


def kernel(item_indices, item_table, user_table):
    tt = item_table.T
    user = user_table.reshape(_D)
    s_sc = _matvec_sc(tt, user)
    s_tc = _matvec_tc(tt, user_table)
    idx = item_indices.astype(jnp.int32).reshape(_NW * _NCH, _CHUNK)
    return _gather_sigmoid(idx, s_sc, s_tc)


# speedup vs baseline: 1.6486x; 1.0838x over previous
import functools

import jax
import jax.numpy as jnp
from jax import lax
from jax.experimental import pallas as pl
from jax.experimental.pallas import tpu as pltpu, tpu_sc as plsc

_B = 16384
_D = 32
_V = 1000000
_BLK = 65536
_NW = 32
_BPW = _B // _NW
_CHUNK = 128
_NCH = _BPW // _CHUNK
_L = 16

_sc_mesh = plsc.VectorSubcoreMesh(core_axis_name="c", subcore_axis_name="s")


def _matvec_body(u_ref, t_ref, s_ref):
    s_ref[...] = jnp.dot(
        u_ref[...], t_ref[...], preferred_element_type=jnp.float32
    ).reshape(_BLK)


def _matvec(tt, user):
    grid = (_V + _BLK - 1) // _BLK
    return pl.pallas_call(
        _matvec_body,
        grid=(grid,),
        in_specs=[
            pl.BlockSpec((1, _D), lambda i: (0, 0)),
            pl.BlockSpec((_D, _BLK), lambda i: (0, i)),
        ],
        out_specs=pl.BlockSpec((_BLK,), lambda i: (i,)),
        out_shape=jax.ShapeDtypeStruct((_V,), jnp.float32),
    )(user, tt)


@functools.partial(
    pl.kernel,
    mesh=_sc_mesh,
    out_type=jax.ShapeDtypeStruct((_B,), jnp.float32),
    compiler_params=pltpu.CompilerParams(
        needs_layout_passes=False, use_tc_tiling_on_sc=False
    ),
    scratch_types=[
        pltpu.VMEM((_NCH, _CHUNK), jnp.int32),
        pltpu.VMEM((_BPW,), jnp.float32),
        pltpu.VMEM((_BPW,), jnp.float32),
        pltpu.SemaphoreType.DMA,
    ],
)
def _gather_sigmoid(idx_hbm, s_hbm, out_hbm, idx_v, g_v, out_v, sem):
    wid = lax.axis_index("s") * 2 + lax.axis_index("c")
    base = wid * _BPW

    pltpu.sync_copy(idx_hbm.at[pl.ds(wid * _NCH, _NCH)], idx_v)

    copies = []
    for j in range(_NCH):
        copies.append(
            pltpu.async_copy(
                s_hbm.at[idx_v.at[j]], g_v.at[pl.ds(j * _CHUNK, _CHUNK)], sem
            )
        )
    for c in copies:
        c.wait()

    def body(g, carry):
        r0 = pl.multiple_of(g * _L, _L)
        x = g_v[pl.ds(r0, _L)]
        out_v[pl.ds(r0, _L)] = 1.0 / (1.0 + jnp.exp(-x))
        return carry

    lax.fori_loop(0, _BPW // _L, body, 0)

    pltpu.sync_copy(out_v, out_hbm.at[pl.ds(base, _BPW)])


def kernel(item_indices, item_table, user_table):
    tt = item_table.T
    s = _matvec(tt, user_table)
    idx = item_indices.astype(jnp.int32).reshape(_NW * _NCH, _CHUNK)
    return _gather_sigmoid(idx, s)
